# baseline (speedup 1.0000x reference)
import jax
import jax.numpy as jnp
from jax import lax
from jax.experimental import pallas as pl
from jax.experimental.pallas import tpu as pltpu
from jax.experimental.pallas import tpu_sc as plsc

_NU = 1000000
_NI = 100000
_D = 32
_B = 16384

_INFO = plsc.get_sparse_core_info()
_NC = _INFO.num_cores
_NS = _INFO.num_subcores
_NW = _NC * _NS
_BPW = _B // _NW
_L = 16


def _body(z_hbm, tu_hbm, ti_hbm, uy_hbm, iy_hbm, out_hbm,
          tu_v, ti_v, ti2_v, zu_v, zi_v, uy_v, iy_v, out_v, sem):
    wid = lax.axis_index("s") * _NC + lax.axis_index("c")
    base = wid * _BPW

    pltpu.sync_copy(tu_hbm.at[pl.ds(base, _BPW)], tu_v)
    pltpu.sync_copy(ti_hbm.at[pl.ds(base, _BPW)], ti_v)

    def shift(i, _):
        ti2_v[pl.ds(i * _L, _L)] = ti_v[pl.ds(i * _L, _L)] + _NU
        return 0

    lax.fori_loop(0, _BPW // _L, shift, 0)

    cp_a = pltpu.async_copy(z_hbm.at[tu_v], zu_v, sem)
    cp_b = pltpu.async_copy(z_hbm.at[ti2_v], zi_v, sem)
    cp_c = pltpu.async_copy(uy_hbm.at[tu_v], uy_v, sem)
    cp_d = pltpu.async_copy(iy_hbm.at[ti_v], iy_v, sem)
    cp_a.wait()
    cp_b.wait()
    cp_c.wait()
    cp_d.wait()

    lane = lax.broadcasted_iota(jnp.int32, (_L,), 0)

    def dot(g, _):
        acc = jnp.zeros((_L,), jnp.float32)
        for k in range(_L):
            j = g * _L + k
            u = (zu_v[j, pl.ds(0, _L)] * zi_v[j, pl.ds(0, _L)]
                 + zu_v[j, pl.ds(_L, _L)] * zi_v[j, pl.ds(_L, _L)]
                 + uy_v[j, pl.ds(0, _L)] * iy_v[j, pl.ds(0, _L)]
                 + uy_v[j, pl.ds(_L, _L)] * iy_v[j, pl.ds(_L, _L)])
            acc = jnp.where(lane == k, jnp.sum(u), acc)
        out_v[pl.ds(g * _L, _L)] = acc
        return 0

    lax.fori_loop(0, _BPW // _L, dot, 0)

    pltpu.sync_copy(out_v, out_hbm.at[pl.ds(base, _BPW)])


@jax.jit
def _mf(z, tu, ti, uy, iy):
    mesh = plsc.VectorSubcoreMesh(core_axis_name="c", subcore_axis_name="s")
    return pl.kernel(
        _body,
        mesh=mesh,
        compiler_params=pltpu.CompilerParams(
            needs_layout_passes=False, use_tc_tiling_on_sc=False),
        out_type=jax.ShapeDtypeStruct((_B,), jnp.float32),
        scratch_types=[
            pltpu.VMEM((_BPW,), jnp.int32),
            pltpu.VMEM((_BPW,), jnp.int32),
            pltpu.VMEM((_BPW,), jnp.int32),
            pltpu.VMEM((_BPW, _D), jnp.float32),
            pltpu.VMEM((_BPW, _D), jnp.float32),
            pltpu.VMEM((_BPW, _D), jnp.float32),
            pltpu.VMEM((_BPW, _D), jnp.float32),
            pltpu.VMEM((_BPW,), jnp.float32),
            pltpu.SemaphoreType.DMA,
        ],
    )(z, tu, ti, uy, iy)


def kernel(Z, Tu, Ti, uY, iY):
    return _mf(Z, Tu.astype(jnp.int32), Ti.astype(jnp.int32), uY, iY)


# speedup vs baseline: 1.4189x; 1.4189x over previous
import jax
import jax.numpy as jnp
from jax import lax
from jax.experimental import pallas as pl
from jax.experimental.pallas import tpu as pltpu
from jax.experimental.pallas import tpu_sc as plsc

_NU = 1000000
_NI = 100000
_D = 32
_B = 16384

_INFO = plsc.get_sparse_core_info()
_NC = _INFO.num_cores
_NS = _INFO.num_subcores
_NW = _NC * _NS
_BPW = _B // _NW
_L = 16
_NCH = _BPW // _L


def _body(z_hbm, tu_hbm, ti_hbm, uy_hbm, iy_hbm, out_hbm,
          tu_v, ti_v, zu_v, zi_v, uy_v, iy_v, out_v, sem):
    wid = lax.axis_index("s") * _NC + lax.axis_index("c")
    base = wid * _BPW

    pltpu.sync_copy(tu_hbm.at[pl.ds(base, _BPW)], tu_v)
    pltpu.sync_copy(ti_hbm.at[pl.ds(base, _BPW)], ti_v)

    lane = lax.broadcasted_iota(jnp.int32, (_L,), 0)

    def chunk(c, _):
        tu16 = tu_v[pl.ds(c * _L, _L)]
        ti16 = ti_v[pl.ds(c * _L, _L)]
        ti16z = ti16 + _NU
        copies = []
        for k in range(_L):
            ru = tu16[k]
            ri = ti16[k]
            riz = ti16z[k]
            copies.append(pltpu.async_copy(z_hbm.at[ru], zu_v.at[k], sem))
            copies.append(pltpu.async_copy(z_hbm.at[riz], zi_v.at[k], sem))
            copies.append(pltpu.async_copy(uy_hbm.at[ru], uy_v.at[k], sem))
            copies.append(pltpu.async_copy(iy_hbm.at[ri], iy_v.at[k], sem))
        for cp in copies:
            cp.wait()
        acc = jnp.zeros((_L,), jnp.float32)
        for k in range(_L):
            u = (zu_v[k, pl.ds(0, _L)] * zi_v[k, pl.ds(0, _L)]
                 + zu_v[k, pl.ds(_L, _L)] * zi_v[k, pl.ds(_L, _L)]
                 + uy_v[k, pl.ds(0, _L)] * iy_v[k, pl.ds(0, _L)]
                 + iy_v[k, pl.ds(_L, _L)] * uy_v[k, pl.ds(_L, _L)])
            acc = jnp.where(lane == k, jnp.sum(u), acc)
        out_v[pl.ds(c * _L, _L)] = acc
        return 0

    lax.fori_loop(0, _NCH, chunk, 0)

    pltpu.sync_copy(out_v, out_hbm.at[pl.ds(base, _BPW)])


@jax.jit
def _mf(z, tu, ti, uy, iy):
    mesh = plsc.VectorSubcoreMesh(core_axis_name="c", subcore_axis_name="s")
    return pl.kernel(
        _body,
        mesh=mesh,
        compiler_params=pltpu.CompilerParams(
            needs_layout_passes=False, use_tc_tiling_on_sc=True),
        out_type=jax.ShapeDtypeStruct((_B,), jnp.float32),
        scratch_types=[
            pltpu.VMEM((_BPW,), jnp.int32),
            pltpu.VMEM((_BPW,), jnp.int32),
            pltpu.VMEM((_L, _D), jnp.float32),
            pltpu.VMEM((_L, _D), jnp.float32),
            pltpu.VMEM((_L, _D), jnp.float32),
            pltpu.VMEM((_L, _D), jnp.float32),
            pltpu.VMEM((_BPW,), jnp.float32),
            pltpu.SemaphoreType.DMA,
        ],
    )(z, tu, ti, uy, iy)


def kernel(Z, Tu, Ti, uY, iY):
    return _mf(Z, Tu.astype(jnp.int32), Ti.astype(jnp.int32), uY, iY)
